# baseline (speedup 1.0000x reference)
import functools
import math

import jax
import jax.numpy as jnp
from jax import lax
from jax.experimental import pallas as pl
from jax.experimental.pallas import tpu as pltpu
from jax.experimental.pallas import tpu_sc as plsc

TOP_K = 10
K_PARA = 1
LEN_K = K_PARA * TOP_K - 1

_INV_DENOM = [1.0 / math.log2(2.0 + t) for t in range(LEN_K)]

_L = 16
_NW = 32


_NG = 16


def _ds16(base):
    return pl.ds(pl.multiple_of(base, _L), _L)


def _sc_topk_body(y_hbm, idx_hbm, rowbuf, mg, ag, foldv, foldi, outb, *,
                  n, rows_pw, grp):
    nc = 2
    wid = lax.axis_index("s") * nc + lax.axis_index("c")
    lane = lax.broadcasted_iota(jnp.int32, (_L,), 0)
    nchunk = n // _L
    cpg = nchunk // _NG
    ngrp = rows_pw // grp
    big = jnp.full((_L,), n * _L, jnp.int32)
    neg2 = jnp.full((_L,), -2.0, jnp.float32)

    foldv[pl.ds(_L, _L)] = jnp.full((_L,), -8.0, jnp.float32)
    foldi[pl.ds(_L, _L)] = big

    def fold_max(v):
        foldv[pl.ds(0, _L)] = v
        for sh in (8, 4, 2):
            a = foldv[pl.ds(0, _L)]
            b = foldv[pl.ds(sh, _L)]
            foldv[pl.ds(0, _L)] = jnp.maximum(a, b)
        a = foldv[pl.ds(0, _L)]
        b = foldv[pl.ds(1, _L)]
        return jnp.maximum(a, b)[0]

    def fold_min_i(v):
        foldi[pl.ds(0, _L)] = v
        for sh in (8, 4, 2):
            a = foldi[pl.ds(0, _L)]
            b = foldi[pl.ds(sh, _L)]
            foldi[pl.ds(0, _L)] = jnp.minimum(a, b)
        a = foldi[pl.ds(0, _L)]
        b = foldi[pl.ds(1, _L)]
        return jnp.minimum(a, b)[0]

    def scan_group(rbase, g):
        def sg(k, carry):
            m, a = carry
            ch = g * cpg + k
            v = rowbuf[_ds16(rbase + ch * _L)]
            gt = v > m
            return jnp.where(gt, v, m), jnp.where(gt, ch, a)

        return lax.fori_loop(0, cpg, sg,
                             (jnp.full((_L,), -8.0, jnp.float32), big))

    def process_row(r, j):
        rbase = j * n
        dch = (r // _L) * _L
        dl = r - dch
        v = rowbuf[_ds16(rbase + dch)]
        rowbuf[_ds16(rbase + dch)] = jnp.where(lane == dl, -1.0, v)

        def pa(g, _):
            m, a = scan_group(rbase, g)
            mg[_ds16(g * _L)] = m
            ag[_ds16(g * _L)] = a
            return 0

        lax.fori_loop(0, _NG, pa, 0)

        outv = big
        for t in range(LEN_K):
            def fm(g, m):
                return jnp.maximum(m, mg[_ds16(g * _L)])

            macc = lax.fori_loop(0, _NG, fm,
                                 jnp.full((_L,), -8.0, jnp.float32))
            m = fold_max(macc)

            def fi(g, acc):
                mv = mg[_ds16(g * _L)]
                av = ag[_ds16(g * _L)]
                gi = av * _L + lane
                return jnp.minimum(acc, jnp.where(mv == m, gi, big))

            iacc = lax.fori_loop(0, _NG, fi, big)
            ix = fold_min_i(iacc)

            ch16 = (ix >> 4) << 4
            il = ix & (_L - 1)
            w = rowbuf[_ds16(rbase + ch16)]
            rowbuf[_ds16(rbase + ch16)] = jnp.where(lane == il, neg2, w)
            gsel = ix >> 8
            m2, a2 = scan_group(rbase, gsel)
            mg[_ds16(gsel * _L)] = m2
            ag[_ds16(gsel * _L)] = a2

            outv = jnp.where(lane == t, ix, outv)

        outb[_ds16(j * _L)] = outv

    def group(gi, _):
        row0 = wid * rows_pw + gi * grp
        pltpu.sync_copy(y_hbm.at[pl.ds(row0 * n, grp * n)], rowbuf)

        def rows(j, _):
            process_row(row0 + j, j)
            return 0

        lax.fori_loop(0, grp, rows, 0)
        pltpu.sync_copy(outb, idx_hbm.at[pl.ds(row0 * _L, grp * _L)])
        return 0

    lax.fori_loop(0, ngrp, group, 0)


def _sc_topk(y):
    n = y.shape[0]
    rows_pw = n // _NW
    grp = 8
    mesh = plsc.VectorSubcoreMesh(core_axis_name="c", subcore_axis_name="s")
    body = functools.partial(_sc_topk_body, n=n, rows_pw=rows_pw, grp=grp)
    f = pl.kernel(
        body,
        out_type=jax.ShapeDtypeStruct((n * _L,), jnp.int32),
        mesh=mesh,
        scratch_types=[
            pltpu.VMEM((grp * n,), jnp.float32),
            pltpu.VMEM((_NG * _L,), jnp.float32),
            pltpu.VMEM((_NG * _L,), jnp.int32),
            pltpu.VMEM((2 * _L,), jnp.float32),
            pltpu.VMEM((2 * _L,), jnp.int32),
            pltpu.VMEM((grp * _L,), jnp.int32),
        ],
    )
    return f(y.reshape(-1)).reshape(n, _L)


def _norm_kernel(o_ref, out_ref):
    o = o_ref[...]
    nrm = jnp.sqrt(jnp.sum(o * o, axis=1, keepdims=True))
    nrm = jnp.where(nrm == 0.0, 1.0, nrm)
    out_ref[...] = o / nrm


def _main_kernel(an_blk_ref, an_full_ref, yidx_ref, out_ref, *, blk, n):
    i = pl.program_id(0)
    an_blk = an_blk_ref[...]
    an_full = an_full_ref[...]

    x = jax.lax.dot_general(
        an_blk, an_full,
        dimension_numbers=(((1,), (1,)), ((), ())),
        preferred_element_type=jnp.float32,
    )
    x = 5.0 * x + 5.0

    col = jax.lax.broadcasted_iota(jnp.int32, (blk, n), 1)
    row = i * blk + jax.lax.broadcasted_iota(jnp.int32, (blk, n), 0)
    diag = col == row

    neg = jnp.float32(-jnp.inf)

    xm = jnp.where(diag, neg, x)
    idcg = jnp.zeros((blk, 1), jnp.float32)
    for t in range(LEN_K):
        m = jnp.max(xm, axis=1, keepdims=True)
        idcg = idcg + (jnp.exp2(m) - 1.0) * _INV_DENOM[t]
        xm = jnp.where(xm == m, neg, xm)

    dcg = jnp.zeros((blk, 1), jnp.float32)
    for t in range(LEN_K):
        sel = col == yidx_ref[:, t:t + 1]
        xg = jnp.max(jnp.where(sel, x, neg), axis=1, keepdims=True)
        dcg = dcg + (jnp.exp2(xg) - 1.0) * _INV_DENOM[t]

    ndcg = dcg / idcg

    @pl.when(i == 0)
    def _():
        out_ref[...] = jnp.zeros((1, 1), jnp.float32)

    out_ref[...] += jnp.sum(ndcg, keepdims=True)


def kernel(output, y_similarity):
    n, d = output.shape

    yidx = _sc_topk(y_similarity)

    a_norm = pl.pallas_call(
        _norm_kernel,
        out_shape=jax.ShapeDtypeStruct((n, d), jnp.float32),
    )(output)

    blk = min(256, n)
    grid = n // blk

    body = functools.partial(_main_kernel, blk=blk, n=n)

    total = pl.pallas_call(
        body,
        grid=(grid,),
        in_specs=[
            pl.BlockSpec((blk, d), lambda i: (i, 0)),
            pl.BlockSpec((n, d), lambda i: (0, 0)),
            pl.BlockSpec((blk, _L), lambda i: (i, 0)),
        ],
        out_specs=pl.BlockSpec((1, 1), lambda i: (0, 0)),
        out_shape=jax.ShapeDtypeStruct((1, 1), jnp.float32),
    )(a_norm, a_norm, yidx)

    return total[0, 0] / n


# speedup vs baseline: 1.0130x; 1.0130x over previous
import functools
import math

import jax
import jax.numpy as jnp
from jax import lax
from jax.experimental import pallas as pl
from jax.experimental.pallas import tpu as pltpu
from jax.experimental.pallas import tpu_sc as plsc

TOP_K = 10
K_PARA = 1
LEN_K = K_PARA * TOP_K - 1

_INV_DENOM = [1.0 / math.log2(2.0 + t) for t in range(LEN_K)]

_L = 16
_NW = 32


_NG = 16


def _ds16(base):
    return pl.ds(pl.multiple_of(base, _L), _L)


def _sc_topk_body(y_hbm, idx_hbm, rowbuf, mg, ag, foldv, foldi, outb, *,
                  n, rows_pw, grp):
    nc = 2
    wid = lax.axis_index("s") * nc + lax.axis_index("c")
    lane = lax.broadcasted_iota(jnp.int32, (_L,), 0)
    nchunk = n // _L
    cpg = nchunk // _NG
    ngrp = rows_pw // grp
    big = jnp.full((_L,), n * _L, jnp.int32)
    neg2 = jnp.full((_L,), -2.0, jnp.float32)

    foldv[pl.ds(_L, _L)] = jnp.full((_L,), -8.0, jnp.float32)
    foldi[pl.ds(_L, _L)] = big

    def fold_max(v):
        foldv[pl.ds(0, _L)] = v
        for sh in (8, 4, 2):
            a = foldv[pl.ds(0, _L)]
            b = foldv[pl.ds(sh, _L)]
            foldv[pl.ds(0, _L)] = jnp.maximum(a, b)
        a = foldv[pl.ds(0, _L)]
        b = foldv[pl.ds(1, _L)]
        return jnp.maximum(a, b)[0]

    def fold_min_i(v):
        foldi[pl.ds(0, _L)] = v
        for sh in (8, 4, 2):
            a = foldi[pl.ds(0, _L)]
            b = foldi[pl.ds(sh, _L)]
            foldi[pl.ds(0, _L)] = jnp.minimum(a, b)
        a = foldi[pl.ds(0, _L)]
        b = foldi[pl.ds(1, _L)]
        return jnp.minimum(a, b)[0]

    def scan_group(rbase, g):
        m = jnp.full((_L,), -8.0, jnp.float32)
        a = big
        for k in range(cpg):
            ch = g * cpg + k
            v = rowbuf[_ds16(rbase + ch * _L)]
            gt = v > m
            m = jnp.where(gt, v, m)
            a = jnp.where(gt, ch, a)
        return m, a

    def process_row(r, j):
        rbase = j * n
        dch = (r // _L) * _L
        dl = r - dch
        v = rowbuf[_ds16(rbase + dch)]
        rowbuf[_ds16(rbase + dch)] = jnp.where(lane == dl, -1.0, v)

        for g in range(_NG):
            m, a = scan_group(rbase, g)
            mg[_ds16(g * _L)] = m
            ag[_ds16(g * _L)] = a

        outv = big
        for t in range(LEN_K):
            macc = jnp.full((_L,), -8.0, jnp.float32)
            for g in range(_NG):
                macc = jnp.maximum(macc, mg[_ds16(g * _L)])
            m = fold_max(macc)

            iacc = big
            for g in range(_NG):
                mv = mg[_ds16(g * _L)]
                av = ag[_ds16(g * _L)]
                gi = av * _L + lane
                iacc = jnp.minimum(iacc, jnp.where(mv == m, gi, big))
            ix = fold_min_i(iacc)

            ch16 = (ix >> 4) << 4
            il = ix & (_L - 1)
            w = rowbuf[_ds16(rbase + ch16)]
            rowbuf[_ds16(rbase + ch16)] = jnp.where(lane == il, neg2, w)
            gsel = ix >> 8
            m2, a2 = scan_group(rbase, gsel)
            mg[_ds16(gsel * _L)] = m2
            ag[_ds16(gsel * _L)] = a2

            outv = jnp.where(lane == t, ix, outv)

        outb[_ds16(j * _L)] = outv

    def group(gi, _):
        row0 = wid * rows_pw + gi * grp
        pltpu.sync_copy(y_hbm.at[pl.ds(row0 * n, grp * n)], rowbuf)

        def rows(j, _):
            process_row(row0 + j, j)
            return 0

        lax.fori_loop(0, grp, rows, 0)
        pltpu.sync_copy(outb, idx_hbm.at[pl.ds(row0 * _L, grp * _L)])
        return 0

    lax.fori_loop(0, ngrp, group, 0)


def _sc_topk(y):
    n = y.shape[0]
    rows_pw = n // _NW
    grp = 8
    mesh = plsc.VectorSubcoreMesh(core_axis_name="c", subcore_axis_name="s")
    body = functools.partial(_sc_topk_body, n=n, rows_pw=rows_pw, grp=grp)
    f = pl.kernel(
        body,
        out_type=jax.ShapeDtypeStruct((n * _L,), jnp.int32),
        mesh=mesh,
        scratch_types=[
            pltpu.VMEM((grp * n,), jnp.float32),
            pltpu.VMEM((_NG * _L,), jnp.float32),
            pltpu.VMEM((_NG * _L,), jnp.int32),
            pltpu.VMEM((2 * _L,), jnp.float32),
            pltpu.VMEM((2 * _L,), jnp.int32),
            pltpu.VMEM((grp * _L,), jnp.int32),
        ],
    )
    return f(y.reshape(-1)).reshape(n, _L)


def _norm_kernel(o_ref, out_ref):
    o = o_ref[...]
    nrm = jnp.sqrt(jnp.sum(o * o, axis=1, keepdims=True))
    nrm = jnp.where(nrm == 0.0, 1.0, nrm)
    out_ref[...] = o / nrm


def _main_kernel(an_blk_ref, an_full_ref, yidx_ref, out_ref, *, blk, n):
    i = pl.program_id(0)
    an_blk = an_blk_ref[...]
    an_full = an_full_ref[...]

    x = jax.lax.dot_general(
        an_blk, an_full,
        dimension_numbers=(((1,), (1,)), ((), ())),
        preferred_element_type=jnp.float32,
    )
    x = 5.0 * x + 5.0

    col = jax.lax.broadcasted_iota(jnp.int32, (blk, n), 1)
    row = i * blk + jax.lax.broadcasted_iota(jnp.int32, (blk, n), 0)
    diag = col == row

    neg = jnp.float32(-jnp.inf)

    xm = jnp.where(diag, neg, x)
    idcg = jnp.zeros((blk, 1), jnp.float32)
    for t in range(LEN_K):
        m = jnp.max(xm, axis=1, keepdims=True)
        idcg = idcg + (jnp.exp2(m) - 1.0) * _INV_DENOM[t]
        xm = jnp.where(xm == m, neg, xm)

    dcg = jnp.zeros((blk, 1), jnp.float32)
    for t in range(LEN_K):
        sel = col == yidx_ref[:, t:t + 1]
        xg = jnp.max(jnp.where(sel, x, neg), axis=1, keepdims=True)
        dcg = dcg + (jnp.exp2(xg) - 1.0) * _INV_DENOM[t]

    ndcg = dcg / idcg

    @pl.when(i == 0)
    def _():
        out_ref[...] = jnp.zeros((1, 1), jnp.float32)

    out_ref[...] += jnp.sum(ndcg, keepdims=True)


def kernel(output, y_similarity):
    n, d = output.shape

    yidx = _sc_topk(y_similarity)

    a_norm = pl.pallas_call(
        _norm_kernel,
        out_shape=jax.ShapeDtypeStruct((n, d), jnp.float32),
    )(output)

    blk = min(256, n)
    grid = n // blk

    body = functools.partial(_main_kernel, blk=blk, n=n)

    total = pl.pallas_call(
        body,
        grid=(grid,),
        in_specs=[
            pl.BlockSpec((blk, d), lambda i: (i, 0)),
            pl.BlockSpec((n, d), lambda i: (0, 0)),
            pl.BlockSpec((blk, _L), lambda i: (i, 0)),
        ],
        out_specs=pl.BlockSpec((1, 1), lambda i: (0, 0)),
        out_shape=jax.ShapeDtypeStruct((1, 1), jnp.float32),
    )(a_norm, a_norm, yidx)

    return total[0, 0] / n
